# baseline (device time: 23007 ns/iter reference)
import jax
import jax.numpy as jnp
from jax import lax
from jax.experimental import pallas as pl
from jax.experimental.pallas import tpu as pltpu

N_DEV = 32
LOG2_N = 5
EPS = 1e-5
GLOBAL_H = 2048
PAD_R, PAD_C = 8, 128


def kernel(x, Wp):
    b, h, w, c = x.shape
    d = Wp.shape[1]
    n_global = GLOBAL_H * w

    def body(x_ref, wp_ref, out_ref, acc_ref, comm_ref, send_sems, recv_sems):
        my = lax.axis_index("i")

        barrier_sem = pltpu.get_barrier_semaphore()
        for s in range(LOG2_N):
            partner = my ^ (1 << s)
            pl.semaphore_signal(
                barrier_sem, inc=1,
                device_id=(partner,), device_id_type=pl.DeviceIdType.MESH,
            )
        pl.semaphore_wait(barrier_sem, LOG2_N)

        xv = x_ref[...].reshape(b, h * w, c)
        s1 = jnp.sum(xv, axis=1)
        s2 = jnp.sum(xv * xv, axis=1)
        acc_ref[...] = jnp.zeros((PAD_R, PAD_C), jnp.float32)
        acc_ref[0:2 * b, 0:c] = jnp.concatenate([s1, s2], axis=0)

        for s in range(LOG2_N):
            partner = my ^ (1 << s)
            rdma = pltpu.make_async_remote_copy(
                src_ref=acc_ref,
                dst_ref=comm_ref.at[s],
                send_sem=send_sems.at[s],
                recv_sem=recv_sems.at[s],
                device_id=(partner,),
                device_id_type=pl.DeviceIdType.MESH,
            )
            rdma.start()
            rdma.wait()
            acc_ref[...] = acc_ref[...] + comm_ref[s]

        tot = acc_ref[0:2 * b, 0:c]
        mean = tot[0:b] / n_global
        var = tot[b:2 * b] / n_global - mean * mean
        inv = lax.rsqrt(var + EPS)
        hn = (xv - mean[:, None, :]) * inv[:, None, :]
        a = hn / (1.0 + jnp.exp(-hn))
        o = jnp.dot(
            a.reshape(b * h * w, c), wp_ref[...],
            preferred_element_type=jnp.float32,
        )
        out_ref[...] = o.reshape(b, h, w, d)

    return pl.pallas_call(
        body,
        out_shape=jax.ShapeDtypeStruct((b, h, w, d), jnp.float32),
        in_specs=[
            pl.BlockSpec(memory_space=pltpu.VMEM),
            pl.BlockSpec(memory_space=pltpu.VMEM),
        ],
        out_specs=pl.BlockSpec(memory_space=pltpu.VMEM),
        scratch_shapes=[
            pltpu.VMEM((PAD_R, PAD_C), jnp.float32),
            pltpu.VMEM((LOG2_N, PAD_R, PAD_C), jnp.float32),
            pltpu.SemaphoreType.DMA((LOG2_N,)),
            pltpu.SemaphoreType.DMA((LOG2_N,)),
        ],
        compiler_params=pltpu.CompilerParams(collective_id=0),
    )(x, Wp)


# device time: 18219 ns/iter; 1.2628x vs baseline; 1.2628x over previous
import jax
import jax.numpy as jnp
from jax import lax
from jax.experimental import pallas as pl
from jax.experimental.pallas import tpu as pltpu

N_DEV = 32
EPS = 1e-5
GLOBAL_H = 2048


def kernel(x, Wp):
    b, h, w, c = x.shape
    d = Wp.shape[1]
    n_global = GLOBAL_H * w

    def body(x_ref, wp_ref, out_ref, acc_ref, comm_ref, send_sems, recv_sems):
        my = lax.axis_index("i")

        comm_ref[...] = jnp.zeros((N_DEV, b, 2 * c), jnp.float32)

        barrier_sem = pltpu.get_barrier_semaphore()
        for off in range(1, N_DEV):
            pl.semaphore_signal(
                barrier_sem, inc=1,
                device_id=(lax.rem(my + off, N_DEV),),
                device_id_type=pl.DeviceIdType.MESH,
            )
        pl.semaphore_wait(barrier_sem, N_DEV - 1)

        xv = x_ref[...].reshape(b, h * w, c)
        s1 = jnp.sum(xv, axis=1)
        s2 = jnp.sum(xv * xv, axis=1)
        acc_ref[...] = jnp.concatenate([s1, s2], axis=1)

        rdmas = []
        for off in range(1, N_DEV):
            tgt = lax.rem(my + off, N_DEV)
            rdma = pltpu.make_async_remote_copy(
                src_ref=acc_ref,
                dst_ref=comm_ref.at[my],
                send_sem=send_sems.at[off - 1],
                recv_sem=recv_sems.at[my],
                device_id=(tgt,),
                device_id_type=pl.DeviceIdType.MESH,
            )
            rdma.start()
            rdmas.append(rdma)

        for j in range(N_DEV):
            @pl.when(j != my)
            def _():
                pltpu.make_async_remote_copy(
                    src_ref=acc_ref,
                    dst_ref=comm_ref.at[j],
                    send_sem=send_sems.at[0],
                    recv_sem=recv_sems.at[j],
                    device_id=(my,),
                    device_id_type=pl.DeviceIdType.MESH,
                ).wait_recv()

        tot = jnp.sum(comm_ref[...], axis=0) + acc_ref[...]
        mean = tot[:, 0:c] / n_global
        var = tot[:, c:2 * c] / n_global - mean * mean
        inv = lax.rsqrt(var + EPS)
        hn = (xv - mean[:, None, :]) * inv[:, None, :]
        a = hn / (1.0 + jnp.exp(-hn))
        o = jnp.dot(
            a.reshape(b * h * w, c), wp_ref[...],
            preferred_element_type=jnp.float32,
        )
        out_ref[...] = o.reshape(b, h, w, d)

        for rdma in rdmas:
            rdma.wait_send()

    return pl.pallas_call(
        body,
        out_shape=jax.ShapeDtypeStruct((b, h, w, d), jnp.float32),
        in_specs=[
            pl.BlockSpec(memory_space=pltpu.VMEM),
            pl.BlockSpec(memory_space=pltpu.VMEM),
        ],
        out_specs=pl.BlockSpec(memory_space=pltpu.VMEM),
        scratch_shapes=[
            pltpu.VMEM((b, 2 * c), jnp.float32),
            pltpu.VMEM((N_DEV, b, 2 * c), jnp.float32),
            pltpu.SemaphoreType.DMA((N_DEV - 1,)),
            pltpu.SemaphoreType.DMA((N_DEV,)),
        ],
        compiler_params=pltpu.CompilerParams(collective_id=0),
    )(x, Wp)


# device time: 17922 ns/iter; 1.2837x vs baseline; 1.0166x over previous
import jax
import jax.numpy as jnp
from jax import lax
from jax.experimental import pallas as pl
from jax.experimental.pallas import tpu as pltpu

N_DEV = 32
EPS = 1e-5
GLOBAL_H = 2048


def kernel(x, Wp):
    b, h, w, c = x.shape
    d = Wp.shape[1]
    n_global = GLOBAL_H * w

    def body(x_ref, wp_ref, out_ref, acc_ref, comm_ref, send_sems, recv_sems):
        my = lax.axis_index("i")

        comm_ref[...] = jnp.zeros((N_DEV, b, 2 * c), jnp.float32)

        barrier_sem = pltpu.get_barrier_semaphore()
        for off in range(1, N_DEV):
            pl.semaphore_signal(
                barrier_sem, inc=1,
                device_id=(lax.rem(my + off, N_DEV),),
                device_id_type=pl.DeviceIdType.MESH,
            )

        xv = x_ref[...].reshape(b, h * w, c)
        s1 = jnp.sum(xv, axis=1)
        s2 = jnp.sum(xv * xv, axis=1)
        acc_ref[...] = jnp.concatenate([s1, s2], axis=1)

        pl.semaphore_wait(barrier_sem, N_DEV - 1)

        rdmas = []
        for off in range(1, N_DEV):
            tgt = lax.rem(my + off, N_DEV)
            rdma = pltpu.make_async_remote_copy(
                src_ref=acc_ref,
                dst_ref=comm_ref.at[my],
                send_sem=send_sems.at[off - 1],
                recv_sem=recv_sems.at[my],
                device_id=(tgt,),
                device_id_type=pl.DeviceIdType.MESH,
            )
            rdma.start()
            rdmas.append(rdma)

        for j in range(N_DEV):
            @pl.when(j != my)
            def _():
                pltpu.make_async_remote_copy(
                    src_ref=acc_ref,
                    dst_ref=comm_ref.at[j],
                    send_sem=send_sems.at[0],
                    recv_sem=recv_sems.at[j],
                    device_id=(my,),
                    device_id_type=pl.DeviceIdType.MESH,
                ).wait_recv()

        tot = jnp.sum(comm_ref[...], axis=0) + acc_ref[...]
        mean = tot[:, 0:c] / n_global
        var = tot[:, c:2 * c] / n_global - mean * mean
        inv = lax.rsqrt(var + EPS)
        hn = (xv - mean[:, None, :]) * inv[:, None, :]
        a = hn / (1.0 + jnp.exp(-hn))
        o = jnp.dot(
            a.reshape(b * h * w, c), wp_ref[...],
            preferred_element_type=jnp.float32,
        )
        out_ref[...] = o.reshape(b, h, w, d)

        for rdma in rdmas:
            rdma.wait_send()

    return pl.pallas_call(
        body,
        out_shape=jax.ShapeDtypeStruct((b, h, w, d), jnp.float32),
        in_specs=[
            pl.BlockSpec(memory_space=pltpu.VMEM),
            pl.BlockSpec(memory_space=pltpu.VMEM),
        ],
        out_specs=pl.BlockSpec(memory_space=pltpu.VMEM),
        scratch_shapes=[
            pltpu.VMEM((b, 2 * c), jnp.float32),
            pltpu.VMEM((N_DEV, b, 2 * c), jnp.float32),
            pltpu.SemaphoreType.DMA((N_DEV - 1,)),
            pltpu.SemaphoreType.DMA((N_DEV,)),
        ],
        compiler_params=pltpu.CompilerParams(collective_id=0),
    )(x, Wp)
